# Initial kernel scaffold; baseline (speedup 1.0000x reference)
import jax
import jax.numpy as jnp
from jax.experimental import pallas as pl


def kernel(z_grid, U):
    raise NotImplementedError("write your pallas kernel here")


# speedup vs baseline: 6.7790x; 6.7790x over previous
import jax
import jax.numpy as jnp
from jax import lax
from jax.experimental import pallas as pl
from jax.experimental.pallas import tpu as pltpu
from jax.experimental.pallas import tpu_sc as plsc

_DD = 1000000
_NC = 2
_NS = 16
_NW = _NC * _NS
_B = 512
_NCH = _B // 128


def _body(z_hbm, u_hbm, out_hbm, z_v, idx0_v, idx1_v, w_v, f0_v, f1_v, sem):
    wid = lax.axis_index("c") * _NS + lax.axis_index("s")
    n = out_hbm.shape[0]
    per_w = n // _NW
    nblk = per_w // _B

    def blk_body(blk, carry):
        base = wid * per_w + blk * _B
        pltpu.sync_copy(z_hbm.at[pl.ds(base, _B)], z_v)

        def idx_body(g, c):
            o = g * 16
            z = z_v[pl.ds(o, 16)]
            z = jnp.minimum(jnp.maximum(z, 0.0), 1.0)
            zi = z * jnp.float32(_DD - 1)
            z0 = zi.astype(jnp.int32)
            w = zi - z0.astype(jnp.float32)
            idx0_v[pl.ds(o, 16)] = z0
            idx1_v[pl.ds(o, 16)] = jnp.minimum(z0 + 1, _DD - 1)
            w_v[pl.ds(o, 16)] = w
            return c

        lax.fori_loop(0, _B // 16, idx_body, 0)

        copies = []
        for c in range(_NCH):
            o = c * 128
            copies.append(pltpu.async_copy(
                u_hbm.at[idx0_v.at[pl.ds(o, 128)]],
                f0_v.at[pl.ds(o, 128)], sem))
            copies.append(pltpu.async_copy(
                u_hbm.at[idx1_v.at[pl.ds(o, 128)]],
                f1_v.at[pl.ds(o, 128)], sem))
        for cp in copies:
            cp.wait()

        def mix_body(g, c):
            wg = w_v[pl.ds(g * 16, 16)]
            i0 = g * 16
            for l in range(16):
                i = i0 + l
                w = lax.broadcast_in_dim(
                    lax.slice(wg, (l,), (l + 1,)), (16,), (0,))
                a0 = f0_v[i, pl.ds(0, 16)]
                b0 = f1_v[i, pl.ds(0, 16)]
                a1 = f0_v[i, pl.ds(16, 16)]
                b1 = f1_v[i, pl.ds(16, 16)]
                f0_v[i, pl.ds(0, 16)] = a0 + w * (b0 - a0)
                f0_v[i, pl.ds(16, 16)] = a1 + w * (b1 - a1)
            return c

        lax.fori_loop(0, _B // 16, mix_body, 0)

        pltpu.sync_copy(f0_v, out_hbm.at[pl.ds(base, _B)])
        return carry

    lax.fori_loop(0, nblk, blk_body, 0)


def kernel(z_grid, U):
    n = z_grid.shape[0] * z_grid.shape[1]
    z_flat = z_grid.reshape(n)
    mesh = plsc.VectorSubcoreMesh(
        core_axis_name="c", subcore_axis_name="s",
        num_cores=_NC, num_subcores=_NS)
    kern = pl.kernel(
        _body,
        out_type=jax.ShapeDtypeStruct((n, U.shape[1]), jnp.float32),
        mesh=mesh,
        scratch_types=[
            pltpu.VMEM((_B,), jnp.float32),
            pltpu.VMEM((_B,), jnp.int32),
            pltpu.VMEM((_B,), jnp.int32),
            pltpu.VMEM((_B,), jnp.float32),
            pltpu.VMEM((_B, U.shape[1]), jnp.float32),
            pltpu.VMEM((_B, U.shape[1]), jnp.float32),
            pltpu.SemaphoreType.DMA,
        ],
        compiler_params=pltpu.CompilerParams(use_tc_tiling_on_sc=False),
    )
    out = kern(z_flat, U)
    return out.reshape(z_grid.shape[0], z_grid.shape[1], U.shape[1])


# speedup vs baseline: 7.9827x; 1.1776x over previous
import jax
import jax.numpy as jnp
from jax import lax
from jax.experimental import pallas as pl
from jax.experimental.pallas import tpu as pltpu
from jax.experimental.pallas import tpu_sc as plsc

_DD = 1000000
_NC = 2
_NS = 16
_NW = _NC * _NS
_B = 512
_NCH = _B // 128


def _body(z_hbm, u_hbm, out_hbm, z_v, idx0_v, idx1_v, w_v, f0_v, f1_v,
          o_v, sem_z, sem_g, sem_o0, sem_o1):
    wid = lax.axis_index("c") * _NS + lax.axis_index("s")
    n = out_hbm.shape[0]
    per_w = n // _NW
    nblk = per_w // _B
    tile_base = wid * per_w
    sem_o = (sem_o0, sem_o1)

    def compute_idx(slot, base):
        def idx_body(g, c):
            o = g * 16
            z = z_v[slot, pl.ds(o, 16)]
            z = jnp.minimum(jnp.maximum(z, 0.0), 1.0)
            zi = z * jnp.float32(_DD - 1)
            z0 = zi.astype(jnp.int32)
            w = zi - z0.astype(jnp.float32)
            idx0_v[slot, pl.ds(o, 16)] = z0
            idx1_v[slot, pl.ds(o, 16)] = jnp.minimum(z0 + 1, _DD - 1)
            w_v[slot, pl.ds(o, 16)] = w
            return c
        lax.fori_loop(0, _B // 16, idx_body, 0)

    def gather_copies(slot):
        copies = []
        for c in range(_NCH):
            o = c * 128
            copies.append(pltpu.make_async_copy(
                u_hbm.at[idx0_v.at[slot, pl.ds(o, 128)]],
                f0_v.at[slot, pl.ds(o, 128)], sem_g))
            copies.append(pltpu.make_async_copy(
                u_hbm.at[idx1_v.at[slot, pl.ds(o, 128)]],
                f1_v.at[slot, pl.ds(o, 128)], sem_g))
        return copies

    def fire_gathers(slot):
        for cp in gather_copies(slot):
            cp.start()

    def mix(slot):
        def mix_body(g, c):
            wg = w_v[slot, pl.ds(g * 16, 16)]
            i0 = g * 16
            for l in range(16):
                i = i0 + l
                w = lax.broadcast_in_dim(
                    lax.slice(wg, (l,), (l + 1,)), (16,), (0,))
                a0 = f0_v[slot, i, pl.ds(0, 16)]
                b0 = f1_v[slot, i, pl.ds(0, 16)]
                a1 = f0_v[slot, i, pl.ds(16, 16)]
                b1 = f1_v[slot, i, pl.ds(16, 16)]
                o_v[slot, i, pl.ds(0, 16)] = a0 + w * (b0 - a0)
                o_v[slot, i, pl.ds(16, 16)] = a1 + w * (b1 - a1)
            return c
        lax.fori_loop(0, _B // 16, mix_body, 0)

    pltpu.sync_copy(z_hbm.at[pl.ds(tile_base, _B)], z_v.at[0])
    compute_idx(0, tile_base)
    fire_gathers(0)

    def blk_body(k, carry):
        p = lax.rem(k, 2)
        base = tile_base + k * _B
        nxt = base + _B

        def half(p, q):
            zcp = pltpu.make_async_copy(
                z_hbm.at[pl.ds(nxt, _B)], z_v.at[q], sem_z)

            @pl.when(k + 1 < nblk)
            def _():
                zcp.start()

            for cp in gather_copies(p):
                cp.wait()

            @pl.when(k + 1 < nblk)
            def _():
                zcp.wait()
                compute_idx(q, nxt)
                fire_gathers(q)

            @pl.when(k >= 2)
            def _():
                pltpu.make_async_copy(
                    o_v.at[p], out_hbm.at[pl.ds(base - 2 * _B, _B)],
                    sem_o[p]).wait()

            mix(p)
            pltpu.make_async_copy(
                o_v.at[p], out_hbm.at[pl.ds(base, _B)], sem_o[p]).start()

        @pl.when(p == 0)
        def _():
            half(0, 1)

        @pl.when(p == 1)
        def _():
            half(1, 0)

        return carry

    lax.fori_loop(0, nblk, blk_body, 0)

    last = tile_base + (nblk - 1) * _B
    pltpu.make_async_copy(
        o_v.at[(nblk - 2) % 2], out_hbm.at[pl.ds(last - _B, _B)],
        sem_o[(nblk - 2) % 2]).wait()
    pltpu.make_async_copy(
        o_v.at[(nblk - 1) % 2], out_hbm.at[pl.ds(last, _B)],
        sem_o[(nblk - 1) % 2]).wait()


def kernel(z_grid, U):
    n = z_grid.shape[0] * z_grid.shape[1]
    z_flat = z_grid.reshape(n)
    mesh = plsc.VectorSubcoreMesh(
        core_axis_name="c", subcore_axis_name="s",
        num_cores=_NC, num_subcores=_NS)
    kern = pl.kernel(
        _body,
        out_type=jax.ShapeDtypeStruct((n, U.shape[1]), jnp.float32),
        mesh=mesh,
        scratch_types=[
            pltpu.VMEM((2, _B), jnp.float32),
            pltpu.VMEM((2, _B), jnp.int32),
            pltpu.VMEM((2, _B), jnp.int32),
            pltpu.VMEM((2, _B), jnp.float32),
            pltpu.VMEM((2, _B, U.shape[1]), jnp.float32),
            pltpu.VMEM((2, _B, U.shape[1]), jnp.float32),
            pltpu.VMEM((2, _B, U.shape[1]), jnp.float32),
            pltpu.SemaphoreType.DMA,
            pltpu.SemaphoreType.DMA,
            pltpu.SemaphoreType.DMA,
            pltpu.SemaphoreType.DMA,
        ],
        compiler_params=pltpu.CompilerParams(use_tc_tiling_on_sc=False),
    )
    out = kern(z_flat, U)
    return out.reshape(z_grid.shape[0], z_grid.shape[1], U.shape[1])


# speedup vs baseline: 7.9851x; 1.0003x over previous
import jax
import jax.numpy as jnp
from jax import lax
from jax.experimental import pallas as pl
from jax.experimental.pallas import tpu as pltpu
from jax.experimental.pallas import tpu_sc as plsc

_DD = 1000000
_NC = 2
_NS = 16
_NW = _NC * _NS
_B = 512
_NCH = _B // 128


def _body(z_hbm, u_hbm, out_hbm, z_v, idx0_v, idx1_v, w_v, f0_v, f1_v,
          o_v, sem_z, sem_g, sem_o0, sem_o1):
    wid = lax.axis_index("c") * _NS + lax.axis_index("s")
    n = out_hbm.shape[0] // 32
    per_w = n // _NW
    nblk = per_w // _B
    tile_base = wid * per_w
    sem_o = (sem_o0, sem_o1)

    def compute_idx(slot, base):
        def idx_body(g, c):
            o = g * 16
            z = z_v[slot, pl.ds(o, 16)]
            z = jnp.minimum(jnp.maximum(z, 0.0), 1.0)
            zi = z * jnp.float32(_DD - 1)
            z0 = zi.astype(jnp.int32)
            w = zi - z0.astype(jnp.float32)
            idx0_v[slot, pl.ds(o, 16)] = z0
            idx1_v[slot, pl.ds(o, 16)] = jnp.minimum(z0 + 1, _DD - 1)
            w_v[slot, pl.ds(o, 16)] = w
            return c
        lax.fori_loop(0, _B // 16, idx_body, 0)

    def gather_copies(slot):
        copies = []
        for c in range(_NCH):
            o = c * 128
            copies.append(pltpu.make_async_copy(
                u_hbm.at[idx0_v.at[slot, pl.ds(o, 128)]],
                f0_v.at[slot, pl.ds(o, 128)], sem_g))
            copies.append(pltpu.make_async_copy(
                u_hbm.at[idx1_v.at[slot, pl.ds(o, 128)]],
                f1_v.at[slot, pl.ds(o, 128)], sem_g))
        return copies

    def fire_gathers(slot):
        for cp in gather_copies(slot):
            cp.start()

    def mix(slot):
        def mix_body(g, c):
            wg = w_v[slot, pl.ds(g * 16, 16)]
            i0 = g * 16
            for l in range(16):
                i = i0 + l
                w = lax.broadcast_in_dim(
                    lax.slice(wg, (l,), (l + 1,)), (16,), (0,))
                a0 = f0_v[slot, i, pl.ds(0, 16)]
                b0 = f1_v[slot, i, pl.ds(0, 16)]
                a1 = f0_v[slot, i, pl.ds(16, 16)]
                b1 = f1_v[slot, i, pl.ds(16, 16)]
                o_v[slot, pl.ds(i * 32, 16)] = a0 + w * (b0 - a0)
                o_v[slot, pl.ds(i * 32 + 16, 16)] = a1 + w * (b1 - a1)
            return c
        lax.fori_loop(0, _B // 16, mix_body, 0)

    pltpu.sync_copy(z_hbm.at[pl.ds(tile_base, _B)], z_v.at[0])
    compute_idx(0, tile_base)
    fire_gathers(0)

    def blk_body(k, carry):
        p = lax.rem(k, 2)
        base = tile_base + k * _B
        nxt = base + _B

        def half(p, q):
            zcp = pltpu.make_async_copy(
                z_hbm.at[pl.ds(nxt, _B)], z_v.at[q], sem_z)

            @pl.when(k + 1 < nblk)
            def _():
                zcp.start()

            for cp in gather_copies(p):
                cp.wait()

            @pl.when(k + 1 < nblk)
            def _():
                zcp.wait()
                compute_idx(q, nxt)
                fire_gathers(q)

            @pl.when(k >= 2)
            def _():
                pltpu.make_async_copy(
                    o_v.at[p],
                    out_hbm.at[pl.ds((base - 2 * _B) * 32, _B * 32)],
                    sem_o[p]).wait()

            mix(p)
            pltpu.make_async_copy(
                o_v.at[p], out_hbm.at[pl.ds(base * 32, _B * 32)],
                sem_o[p]).start()

        @pl.when(p == 0)
        def _():
            half(0, 1)

        @pl.when(p == 1)
        def _():
            half(1, 0)

        return carry

    lax.fori_loop(0, nblk, blk_body, 0)

    last = tile_base + (nblk - 1) * _B
    pltpu.make_async_copy(
        o_v.at[(nblk - 2) % 2], out_hbm.at[pl.ds((last - _B) * 32, _B * 32)],
        sem_o[(nblk - 2) % 2]).wait()
    pltpu.make_async_copy(
        o_v.at[(nblk - 1) % 2], out_hbm.at[pl.ds(last * 32, _B * 32)],
        sem_o[(nblk - 1) % 2]).wait()


def kernel(z_grid, U):
    n = z_grid.shape[0] * z_grid.shape[1]
    z_flat = z_grid.reshape(n)
    mesh = plsc.VectorSubcoreMesh(
        core_axis_name="c", subcore_axis_name="s",
        num_cores=_NC, num_subcores=_NS)
    kern = pl.kernel(
        _body,
        out_type=jax.ShapeDtypeStruct((n * U.shape[1],), jnp.float32),
        mesh=mesh,
        scratch_types=[
            pltpu.VMEM((2, _B), jnp.float32),
            pltpu.VMEM((2, _B), jnp.int32),
            pltpu.VMEM((2, _B), jnp.int32),
            pltpu.VMEM((2, _B), jnp.float32),
            pltpu.VMEM((2, _B, U.shape[1]), jnp.float32),
            pltpu.VMEM((2, _B, U.shape[1]), jnp.float32),
            pltpu.VMEM((2, _B * U.shape[1]), jnp.float32),
            pltpu.SemaphoreType.DMA,
            pltpu.SemaphoreType.DMA,
            pltpu.SemaphoreType.DMA,
            pltpu.SemaphoreType.DMA,
        ],
        compiler_params=pltpu.CompilerParams(use_tc_tiling_on_sc=False),
    )
    out = kern(z_flat, U)
    return out.reshape(z_grid.shape[0], z_grid.shape[1], U.shape[1])
